# baseline (speedup 1.0000x reference)
import jax
import jax.numpy as jnp
from jax.experimental import pallas as pl
from jax.experimental.pallas import tpu as pltpu

BATCH = 1024
INPUT_DIM = 2048
NUM_GROUPS = 8
GROUP_SIZE = 256
TB = 128

_MSB = -2147483648


def _gumbel_noise():
    nkey = jax.random.fold_in(jax.random.key(42), 7)
    u = jax.random.uniform(nkey, (NUM_GROUPS, INPUT_DIM), dtype=jnp.float32,
                           minval=1e-7, maxval=1.0 - 1e-7)
    return -jnp.log(-jnp.log(u))


def _fused_kernel(x_ref, probs_ref, grouped_ref, mask_ref):
    @pl.when(pl.program_id(0) == 0)
    def _compute_mask():
        msb = jnp.int32(_MSB)
        z = probs_ref[...]
        b = jax.lax.bitcast_convert_type(z, jnp.int32)
        s = jnp.where(b >= 0, b, b ^ jnp.int32(0x7FFFFFFF))
        ones = jnp.ones((INPUT_DIM, 1), jnp.float32)
        kf = jnp.float32(GROUP_SIZE)

        tu = jnp.zeros((NUM_GROUPS, 1), jnp.int32)
        cnt_acc = jnp.full((NUM_GROUPS, 1), float(INPUT_DIM), jnp.float32)
        for bit in range(31, -1, -2):
            b_hi = msb if bit == 31 else jnp.int32(1 << bit)
            b_lo = jnp.int32(1 << (bit - 1))
            c11 = tu | b_hi | b_lo
            c10 = tu | b_hi
            c01 = tu | b_lo
            m = jnp.concatenate(
                [(s >= (c11 ^ msb)).astype(jnp.float32),
                 (s >= (c10 ^ msb)).astype(jnp.float32),
                 (s >= (c01 ^ msb)).astype(jnp.float32)], axis=0)
            n = jax.lax.dot_general(
                m, ones, (((1,), (0,)), ((), ())),
                preferred_element_type=jnp.float32)
            n11, n10, n01 = n[:NUM_GROUPS], n[NUM_GROUPS:2 * NUM_GROUPS], \
                n[2 * NUM_GROUPS:]
            tu = jnp.where(n11 >= kf, c11,
                           jnp.where(n10 >= kf, c10,
                                     jnp.where(n01 >= kf, c01, tu)))
            cnt_acc = jnp.where(n11 >= kf, n11,
                                jnp.where(n10 >= kf, n10,
                                          jnp.where(n01 >= kf, n01, cnt_acc)))
        t_s = tu ^ msb

        @pl.when(jnp.all(cnt_acc == kf))
        def _no_ties():
            mask_ref[...] = (s >= t_s).astype(jnp.float32)

        @pl.when(jnp.logical_not(jnp.all(cnt_acc == kf)))
        def _break_ties():
            gt = s > t_s
            cnt_gt = jnp.sum(gt.astype(jnp.int32), axis=-1, keepdims=True)
            need_eq = GROUP_SIZE - cnt_gt
            eq = s == t_s
            idx = jax.lax.broadcasted_iota(
                jnp.int32, (NUM_GROUPS, INPUT_DIM), 1)
            lo = jnp.zeros((NUM_GROUPS, 1), jnp.int32)
            hi = jnp.full((NUM_GROUPS, 1), INPUT_DIM - 1, jnp.int32)
            for _ in range(11):
                mid = (lo + hi) // 2
                c = jnp.sum((eq & (idx <= mid)).astype(jnp.int32), axis=-1,
                            keepdims=True)
                take = c >= need_eq
                hi = jnp.where(take, mid, hi)
                lo = jnp.where(take, lo, mid + 1)
            mask_ref[...] = (gt | (eq & (idx <= lo))).astype(jnp.float32)

    grouped_ref[...] = mask_ref[...][:, None, :] * x_ref[...][None, :, :]


def kernel(x, group_logits):
    probs = jax.nn.softmax((group_logits + _gumbel_noise()) / 1.0, axis=-1)
    grouped, mask = pl.pallas_call(
        _fused_kernel,
        grid=(BATCH // TB,),
        in_specs=[
            pl.BlockSpec((TB, INPUT_DIM), lambda i: (i, 0)),
            pl.BlockSpec((NUM_GROUPS, INPUT_DIM), lambda i: (0, 0)),
        ],
        out_specs=[
            pl.BlockSpec((NUM_GROUPS, TB, INPUT_DIM), lambda i: (0, i, 0)),
            pl.BlockSpec((NUM_GROUPS, INPUT_DIM), lambda i: (0, 0)),
        ],
        out_shape=[
            jax.ShapeDtypeStruct((NUM_GROUPS, BATCH, INPUT_DIM), jnp.float32),
            jax.ShapeDtypeStruct((NUM_GROUPS, INPUT_DIM), jnp.float32),
        ],
        compiler_params=pltpu.CompilerParams(
            dimension_semantics=("arbitrary",),
        ),
    )(x, probs)
    return (grouped, mask)


# speedup vs baseline: 1.0805x; 1.0805x over previous
import jax
import jax.numpy as jnp
from jax.experimental import pallas as pl
from jax.experimental.pallas import tpu as pltpu

BATCH = 1024
INPUT_DIM = 2048
NUM_GROUPS = 8
GROUP_SIZE = 256
TB = 128

_MSB = -2147483648


def _gumbel_noise():
    nkey = jax.random.fold_in(jax.random.key(42), 7)
    u = jax.random.uniform(nkey, (NUM_GROUPS, INPUT_DIM), dtype=jnp.float32,
                           minval=1e-7, maxval=1.0 - 1e-7)
    return -jnp.log(-jnp.log(u))


def _fused_kernel(x_ref, probs_ref, grouped_ref, mask_ref):
    @pl.when(pl.program_id(0) == 0)
    def _compute_mask():
        msb = jnp.int32(_MSB)
        z = probs_ref[...]
        b = jax.lax.bitcast_convert_type(z, jnp.int32)
        s = jnp.where(b >= 0, b, b ^ jnp.int32(0x7FFFFFFF))
        kk = jnp.int32(GROUP_SIZE)

        bps = 4
        tu = jnp.zeros((NUM_GROUPS, 1), jnp.int32)
        cnt_acc = jnp.full((NUM_GROUPS, 1), INPUT_DIM, jnp.int32)
        for low in range(32 - bps, -1, -bps):
            cands, cnts = [tu], [cnt_acc]
            for v in range(1, 1 << bps):
                shifted = v << low
                if shifted >= 2 ** 31:
                    shifted -= 2 ** 32
                cand = tu | jnp.int32(shifted)
                n = jnp.sum((s >= (cand ^ msb)).astype(jnp.int32), axis=-1,
                            keepdims=True)
                ok = n >= kk
                cands.append(jnp.where(ok, cand, tu))
                cnts.append(jnp.where(ok, n, cnt_acc))
            while len(cands) > 1:
                nc, nn = [], []
                for i in range(0, len(cands) - 1, 2):
                    take = (cands[i + 1] ^ msb) >= (cands[i] ^ msb)
                    nc.append(jnp.where(take, cands[i + 1], cands[i]))
                    nn.append(jnp.where(take, cnts[i + 1], cnts[i]))
                if len(cands) % 2:
                    nc.append(cands[-1])
                    nn.append(cnts[-1])
                cands, cnts = nc, nn
            tu, cnt_acc = cands[0], cnts[0]
        t_s = tu ^ msb

        @pl.when(jnp.all(cnt_acc == kk))
        def _no_ties():
            mask_ref[...] = (s >= t_s).astype(jnp.float32)

        @pl.when(jnp.logical_not(jnp.all(cnt_acc == kk)))
        def _break_ties():
            gt = s > t_s
            cnt_gt = jnp.sum(gt.astype(jnp.int32), axis=-1, keepdims=True)
            need_eq = GROUP_SIZE - cnt_gt
            eq = s == t_s
            idx = jax.lax.broadcasted_iota(
                jnp.int32, (NUM_GROUPS, INPUT_DIM), 1)
            lo = jnp.zeros((NUM_GROUPS, 1), jnp.int32)
            hi = jnp.full((NUM_GROUPS, 1), INPUT_DIM - 1, jnp.int32)
            for _ in range(11):
                mid = (lo + hi) // 2
                c = jnp.sum((eq & (idx <= mid)).astype(jnp.int32), axis=-1,
                            keepdims=True)
                take = c >= need_eq
                hi = jnp.where(take, mid, hi)
                lo = jnp.where(take, lo, mid + 1)
            mask_ref[...] = (gt | (eq & (idx <= lo))).astype(jnp.float32)

    grouped_ref[...] = mask_ref[...][:, None, :] * x_ref[...][None, :, :]


def kernel(x, group_logits):
    probs = jax.nn.softmax((group_logits + _gumbel_noise()) / 1.0, axis=-1)
    grouped, mask = pl.pallas_call(
        _fused_kernel,
        grid=(BATCH // TB,),
        in_specs=[
            pl.BlockSpec((TB, INPUT_DIM), lambda i: (i, 0)),
            pl.BlockSpec((NUM_GROUPS, INPUT_DIM), lambda i: (0, 0)),
        ],
        out_specs=[
            pl.BlockSpec((NUM_GROUPS, TB, INPUT_DIM), lambda i: (0, i, 0)),
            pl.BlockSpec((NUM_GROUPS, INPUT_DIM), lambda i: (0, 0)),
        ],
        out_shape=[
            jax.ShapeDtypeStruct((NUM_GROUPS, BATCH, INPUT_DIM), jnp.float32),
            jax.ShapeDtypeStruct((NUM_GROUPS, INPUT_DIM), jnp.float32),
        ],
        compiler_params=pltpu.CompilerParams(
            dimension_semantics=("arbitrary",),
        ),
    )(x, probs)
    return (grouped, mask)
